# baseline (speedup 1.0000x reference)
import jax
import jax.numpy as jnp
from jax import lax
from jax.experimental import pallas as pl
from jax.experimental.pallas import tpu as pltpu

_SMOOTHING = 0.1
_CONFIDENCE = 1.0 - _SMOOTHING
_V = 100000
_B = 1024
_FILL = _SMOOTHING / (_V - 2)

_VR = 1000
_NBLK = _V // _VR
_DEPTH = 4
_SLAB = 8


def _smooth_body(ids_hbm, out_hbm, ids_v, b0, b1, b2, b3, s0, s1, s2, s3,
                 sem_in):
    bufs = (b0, b1, b2, b3)
    sems = (s0, s1, s2, s3)
    ids_cp = pltpu.make_async_copy(ids_hbm, ids_v, sem_in)
    ids_cp.start()
    ids_cp.wait()
    ids_b = jnp.broadcast_to(ids_v[0:1, :], (_SLAB, _B))

    def compute_block(buf, j):
        def slab(s, carry):
            vocab = lax.broadcasted_iota(jnp.int32, (_SLAB, _B), 0) + (
                j * _VR + s * _SLAB)
            val = jnp.where(vocab == ids_b, _CONFIDENCE, _FILL)
            buf[pl.ds(s * _SLAB, _SLAB), :] = jnp.where(vocab == 0, 0.0, val)
            return carry

        lax.fori_loop(0, _VR // _SLAB, slab, 0)

    for j in range(_NBLK):
        buf, sem = bufs[j % _DEPTH], sems[j % _DEPTH]
        if j >= _DEPTH:
            pltpu.make_async_copy(
                buf, out_hbm.at[pl.ds((j - _DEPTH) * _VR, _VR), :], sem
            ).wait()
        compute_block(buf, j)
        pltpu.make_async_copy(
            buf, out_hbm.at[pl.ds(j * _VR, _VR), :], sem).start()
    for k in range(_DEPTH):
        j = _NBLK - _DEPTH + k
        pltpu.make_async_copy(
            bufs[j % _DEPTH], out_hbm.at[pl.ds(j * _VR, _VR), :],
            sems[j % _DEPTH]).wait()


def kernel(tgt_tok_id):
    ids = tgt_tok_id.reshape(1, _B).astype(jnp.int32)
    out_t = pl.pallas_call(
        _smooth_body,
        in_specs=[pl.BlockSpec(memory_space=pl.ANY)],
        out_specs=pl.BlockSpec(memory_space=pl.ANY),
        out_shape=jax.ShapeDtypeStruct((_V, _B), jnp.float32),
        scratch_shapes=[
            pltpu.VMEM((1, _B), jnp.int32),
            pltpu.VMEM((_VR, _B), jnp.float32),
            pltpu.VMEM((_VR, _B), jnp.float32),
            pltpu.VMEM((_VR, _B), jnp.float32),
            pltpu.VMEM((_VR, _B), jnp.float32),
            pltpu.SemaphoreType.DMA,
            pltpu.SemaphoreType.DMA,
            pltpu.SemaphoreType.DMA,
            pltpu.SemaphoreType.DMA,
            pltpu.SemaphoreType.DMA,
        ],
    )(ids)
    return jnp.transpose(out_t)
